# Initial kernel scaffold; baseline (speedup 1.0000x reference)
import jax
import jax.numpy as jnp
from jax.experimental import pallas as pl


def kernel(x, edge_index, edge_attr, batch, assignment_index_3, iso_type_3, edge_index_3, batch_3, params):
    raise NotImplementedError("write your pallas kernel here")


# speedup vs baseline: 1.7625x; 1.7625x over previous
import functools

import jax
import jax.numpy as jnp
from jax import lax
from jax.experimental import pallas as pl
from jax.experimental.pallas import tpu as pltpu
from jax.experimental.pallas import tpu_sc as plsc

N = 10000
E = 20000
B = 512
N3 = 15000
A = 45000
E3 = 60000

NW = 32
CHUNK = 128

N_PAD = 10112
N3_PAD = 15104
B_PAD = 640
E_PAD = 20480
A_PAD = 45056
E3_PAD = 61440
NV_PAD = 12288
N3V_PAD = 16384


def _mesh():
    return plsc.VectorSubcoreMesh(core_axis_name="c", subcore_axis_name="s")


def _sc_gather(table, idx2, d):
    ep = idx2.shape[0] * idx2.shape[1] * CHUNK
    nch = ep // (NW * CHUNK)
    ept = ep // NW

    def body(tab_hbm, idx_hbm, out_hbm, idxv, rowsv, sem):
        c = lax.axis_index("c")
        s = lax.axis_index("s")
        wid = s * 2 + c
        pltpu.sync_copy(idx_hbm.at[wid], idxv)
        for j in range(nch):
            pltpu.async_copy(tab_hbm.at[idxv.at[j]],
                             rowsv.at[pl.ds(j * CHUNK, CHUNK)], sem).wait()
        pltpu.sync_copy(rowsv, out_hbm.at[pl.ds(wid * ept, ept)])

    f = pl.kernel(
        body,
        out_type=jax.ShapeDtypeStruct((ep, d), jnp.float32),
        mesh=_mesh(),
        compiler_params=pltpu.CompilerParams(use_tc_tiling_on_sc=False),
        scratch_types=[
            pltpu.VMEM((nch, CHUNK), jnp.int32),
            pltpu.VMEM((ept, d), jnp.float32),
            pltpu.SemaphoreType.DMA,
        ],
    )
    return f(table, idx2)


def _sc_scatter_add(vals, idx2, n_acc, d):
    ep = vals.shape[0]
    nch = ep // (NW * CHUNK)
    ept = ep // NW
    zr = n_acc // 16
    zeros = jnp.zeros((zr, d), jnp.float32)

    def body(vals_hbm, idx_hbm, z_hbm, out_hbm, idxv, valsv, acc):
        c = lax.axis_index("c")
        s = lax.axis_index("s")
        wid = s * 2 + c
        pltpu.sync_copy(z_hbm, acc.at[pl.ds(s * zr, zr)])
        plsc.subcore_barrier()
        pltpu.sync_copy(idx_hbm.at[wid], idxv)
        pltpu.sync_copy(vals_hbm.at[pl.ds(wid * ept, ept)], valsv)
        for j in range(nch):
            pltpu.sync_copy(valsv.at[pl.ds(j * CHUNK, CHUNK)],
                            acc.at[idxv.at[j]], add=True)
        plsc.subcore_barrier()
        pltpu.sync_copy(acc.at[pl.ds(s * zr, zr)],
                        out_hbm.at[pl.ds(c * n_acc + s * zr, zr)])

    f = pl.kernel(
        body,
        out_type=jax.ShapeDtypeStruct((2 * n_acc, d), jnp.float32),
        mesh=_mesh(),
        compiler_params=pltpu.CompilerParams(use_tc_tiling_on_sc=False),
        scratch_types=[
            pltpu.VMEM((nch, CHUNK), jnp.int32),
            pltpu.VMEM((ept, d), jnp.float32),
            pltpu.VMEM_SHARED((n_acc, d), jnp.float32),
        ],
    )
    return f(vals, idx2, zeros).reshape(2, n_acc, d)


def _sc_gather_scatter_add(table, src2, dst2, n_acc, d):
    ep = src2.shape[0] * src2.shape[1] * CHUNK
    nch = ep // (NW * CHUNK)
    zr = n_acc // 16
    zeros = jnp.zeros((zr, d), jnp.float32)

    def body(tab_hbm, s_hbm, d_hbm, z_hbm, out_hbm, sidx, didx, rowsv, acc, sem):
        c = lax.axis_index("c")
        s = lax.axis_index("s")
        wid = s * 2 + c
        pltpu.sync_copy(z_hbm, acc.at[pl.ds(s * zr, zr)])
        plsc.subcore_barrier()
        pltpu.sync_copy(s_hbm.at[wid], sidx)
        pltpu.sync_copy(d_hbm.at[wid], didx)
        for j in range(nch):
            pltpu.async_copy(tab_hbm.at[sidx.at[j]], rowsv, sem).wait()
            pltpu.sync_copy(rowsv, acc.at[didx.at[j]], add=True)
        plsc.subcore_barrier()
        pltpu.sync_copy(acc.at[pl.ds(s * zr, zr)],
                        out_hbm.at[pl.ds(c * n_acc + s * zr, zr)])

    f = pl.kernel(
        body,
        out_type=jax.ShapeDtypeStruct((2 * n_acc, d), jnp.float32),
        mesh=_mesh(),
        compiler_params=pltpu.CompilerParams(use_tc_tiling_on_sc=False),
        scratch_types=[
            pltpu.VMEM((nch, CHUNK), jnp.int32),
            pltpu.VMEM((nch, CHUNK), jnp.int32),
            pltpu.VMEM((CHUNK, d), jnp.float32),
            pltpu.VMEM_SHARED((n_acc, d), jnp.float32),
            pltpu.SemaphoreType.DMA,
        ],
    )
    return f(table, src2, dst2, zeros).reshape(2, n_acc, d)


def _tc_edge_messages(ea, xs, w1, b1, w2, b2, mi, mo):
    ep = ea.shape[0]
    be = 512
    grid = ep // be

    def body(ea_ref, xs_ref, w1_ref, b1_ref, w2_ref, b2_ref, out_ref):
        g = jnp.maximum(
            jnp.dot(ea_ref[:], w1_ref[:], preferred_element_type=jnp.float32)
            + b1_ref[:], 0.0)
        g2 = jnp.dot(g, w2_ref[:], preferred_element_type=jnp.float32) + b2_ref[:]
        xsv = xs_ref[:]
        acc = xsv[:, 0:1] * g2[:, 0:mo]
        for i in range(1, mi):
            acc = acc + xsv[:, i:i + 1] * g2[:, i * mo:(i + 1) * mo]
        out_ref[:] = acc

    return pl.pallas_call(
        body,
        grid=(grid,),
        in_specs=[
            pl.BlockSpec((be, 8), lambda i: (i, 0)),
            pl.BlockSpec((be, mi), lambda i: (i, 0)),
            pl.BlockSpec((8, 128), lambda i: (0, 0)),
            pl.BlockSpec((1, 128), lambda i: (0, 0)),
            pl.BlockSpec((128, mi * mo), lambda i: (0, 0)),
            pl.BlockSpec((1, mi * mo), lambda i: (0, 0)),
        ],
        out_specs=pl.BlockSpec((be, mo), lambda i: (i, 0)),
        out_shape=jax.ShapeDtypeStruct((ep, mo), jnp.float32),
    )(ea, xs, w1, b1, w2, b2)


def _tc_nnconv_combine(agg, hprev, root, bias):
    np_, mi = hprev.shape
    mo = root.shape[1]
    bn = 1024
    grid = pl.cdiv(np_, bn)

    def body(a_ref, h_ref, r_ref, b_ref, o_ref):
        o_ref[:] = jnp.maximum(
            a_ref[0] + a_ref[1]
            + jnp.dot(h_ref[:], r_ref[:], preferred_element_type=jnp.float32)
            + b_ref[:], 0.0)

    return pl.pallas_call(
        body,
        grid=(grid,),
        in_specs=[
            pl.BlockSpec((2, bn, mo), lambda i: (0, i, 0)),
            pl.BlockSpec((bn, mi), lambda i: (i, 0)),
            pl.BlockSpec((mi, mo), lambda i: (0, 0)),
            pl.BlockSpec((1, mo), lambda i: (0, 0)),
        ],
        out_specs=pl.BlockSpec((bn, mo), lambda i: (i, 0)),
        out_shape=jax.ShapeDtypeStruct((np_, mo), jnp.float32),
    )(agg, hprev, root, bias)


def _tc_pool_finalize(pooled, cnt, iso):
    np_ = iso.shape[0]
    bn = 1024
    grid = pl.cdiv(np_, bn)

    def body(p_ref, c_ref, iso_ref, o_ref):
        c = c_ref[0][:, 0:1] + c_ref[1][:, 0:1]
        pool = (p_ref[0] + p_ref[1]) / jnp.maximum(c, 1.0)
        o_ref[:] = jnp.concatenate([pool, iso_ref[:]], axis=1)

    return pl.pallas_call(
        body,
        grid=(grid,),
        in_specs=[
            pl.BlockSpec((2, bn, 64), lambda i: (0, i, 0)),
            pl.BlockSpec((2, bn, 16), lambda i: (0, i, 0)),
            pl.BlockSpec((bn, 64), lambda i: (i, 0)),
        ],
        out_specs=pl.BlockSpec((bn, 128), lambda i: (i, 0)),
        out_shape=jax.ShapeDtypeStruct((np_, 128), jnp.float32),
    )(pooled, cnt, iso)


def _tc_gc_combine(aggs, h, wrel, brel, wroot):
    np_, mi = h.shape
    mo = wrel.shape[1]
    bn = 1024
    grid = pl.cdiv(np_, bn)

    def body(*refs):
        a_refs = refs[:len(aggs)]
        h_ref, wr_ref, br_ref, wo_ref, o_ref = refs[len(aggs):]
        a = jnp.concatenate([r[0] + r[1] for r in a_refs], axis=1)
        o_ref[:] = jnp.maximum(
            jnp.dot(a, wr_ref[:], preferred_element_type=jnp.float32)
            + br_ref[:]
            + jnp.dot(h_ref[:], wo_ref[:], preferred_element_type=jnp.float32),
            0.0)

    return pl.pallas_call(
        body,
        grid=(grid,),
        in_specs=[
            pl.BlockSpec((2, bn, a.shape[2]), lambda i: (0, i, 0))
            for a in aggs
        ] + [
            pl.BlockSpec((bn, mi), lambda i: (i, 0)),
            pl.BlockSpec((mi, mo), lambda i: (0, 0)),
            pl.BlockSpec((1, mo), lambda i: (0, 0)),
            pl.BlockSpec((mi, mo), lambda i: (0, 0)),
        ],
        out_specs=pl.BlockSpec((bn, mo), lambda i: (i, 0)),
        out_shape=jax.ShapeDtypeStruct((np_, mo), jnp.float32),
    )(*aggs, h, wrel, brel, wroot)


def _tc_head(x1p, x3p, w1, b1, w2, b2, w3, b3):
    def body(x1_ref, x3_ref, w1_ref, b1_ref, w2_ref, b2_ref, w3_ref, b3_ref,
             o_ref):
        z = jnp.concatenate([x1_ref[0] + x1_ref[1], x3_ref[0] + x3_ref[1]],
                            axis=1)
        z = jnp.maximum(
            jnp.dot(z, w1_ref[:], preferred_element_type=jnp.float32)
            + b1_ref[:], 0.0)
        z = jnp.maximum(
            jnp.dot(z, w2_ref[:], preferred_element_type=jnp.float32)
            + b2_ref[:], 0.0)
        o_ref[:] = (jnp.dot(z, w3_ref[:], preferred_element_type=jnp.float32)
                    + b3_ref[:])

    return pl.pallas_call(
        body,
        grid=(1,),
        in_specs=[
            pl.BlockSpec((2, B, 64), lambda i: (0, 0, 0)),
            pl.BlockSpec((2, B, 64), lambda i: (0, 0, 0)),
            pl.BlockSpec((128, 64), lambda i: (0, 0)),
            pl.BlockSpec((1, 64), lambda i: (0, 0)),
            pl.BlockSpec((64, 32), lambda i: (0, 0)),
            pl.BlockSpec((1, 32), lambda i: (0, 0)),
            pl.BlockSpec((32, 8), lambda i: (0, 0)),
            pl.BlockSpec((1, 8), lambda i: (0, 0)),
        ],
        out_specs=pl.BlockSpec((B, 8), lambda i: (0, 0)),
        out_shape=jax.ShapeDtypeStruct((B, 8), jnp.float32),
    )(x1p, x3p, w1, b1, w2, b2, w3, b3)


def _pad_rows(a, n):
    return jnp.pad(a, ((0, n - a.shape[0]), (0, 0)))


def _pad_idx2(idx, ep, fill):
    return jnp.pad(idx, (0, ep - idx.shape[0]),
                   constant_values=fill).reshape(NW, ep // (NW * CHUNK), CHUNK)


def kernel(x, edge_index, edge_attr, batch, assignment_index_3, iso_type_3,
           edge_index_3, batch_3, params):
    p = params

    x_p = _pad_rows(x, N_PAD)
    iso_p = _pad_rows(iso_type_3, N3_PAD)
    ea_p = jnp.pad(edge_attr, ((0, E_PAD - E), (0, 1)))

    src2 = _pad_idx2(edge_index[0], E_PAD, 0)
    dst2 = _pad_idx2(edge_index[1], E_PAD, N)
    row2 = _pad_idx2(assignment_index_3[0], A_PAD, 0)
    col2 = _pad_idx2(assignment_index_3[1], A_PAD, N3)
    src3_2 = _pad_idx2(edge_index_3[0], E3_PAD, 0)
    dst3_2 = _pad_idx2(edge_index_3[1], E3_PAD, N3)
    batch2 = _pad_idx2(batch, NV_PAD, B)
    batch3_2 = _pad_idx2(batch_3, N3V_PAD, B)

    def r2(b):
        return b.reshape(1, -1)

    h = x_p
    specs = [
        (16, 32, 'nn1', 'conv1'),
        (32, 64, 'nn2', 'conv2'),
        (64, 64, 'nn3', 'conv3'),
    ]
    for mi, mo, nn, conv in specs:
        w1 = jnp.pad(p[nn + '_W1'], ((0, 1), (0, 0)))
        xs = _sc_gather(h, src2, mi)
        msg = _tc_edge_messages(ea_p, xs, w1, r2(p[nn + '_b1']),
                                p[nn + '_W2'], r2(p[nn + '_b2']), mi, mo)
        agg = _sc_scatter_add(msg, dst2, N_PAD, mo)
        h = _tc_nnconv_combine(agg, h, p[conv + '_root'],
                               r2(p[conv + '_bias']))

    x1p = _sc_scatter_add(_pad_rows(h, NV_PAD), batch2, B_PAD, 64)[:, :B]

    pooled = _sc_gather_scatter_add(h, row2, col2, N3_PAD, 64)
    ones16 = jnp.ones((A_PAD, 16), jnp.float32)
    cnt = _sc_scatter_add(ones16, col2, N3_PAD, 16)
    h3 = _tc_pool_finalize(pooled, cnt, iso_p)

    agg_lo = _sc_gather_scatter_add(h3[:, :64], src3_2, dst3_2, N3_PAD, 64)
    agg_hi = _sc_gather_scatter_add(h3[:, 64:], src3_2, dst3_2, N3_PAD, 64)
    h3 = _tc_gc_combine([agg_lo, agg_hi], h3, p['conv6_Wrel'],
                        r2(p['conv6_brel']), p['conv6_Wroot'])
    agg3 = _sc_gather_scatter_add(h3, src3_2, dst3_2, N3_PAD, 64)
    h3 = _tc_gc_combine([agg3], h3, p['conv7_Wrel'], r2(p['conv7_brel']),
                        p['conv7_Wroot'])

    x3p = _sc_scatter_add(_pad_rows(h3, N3V_PAD), batch3_2, B_PAD, 64)[:, :B]

    w3 = jnp.pad(p['fc3_W'], ((0, 0), (0, 7)))
    b3 = jnp.pad(p['fc3_b'], (0, 7))
    z = _tc_head(x1p, x3p, p['fc1_W'], r2(p['fc1_b']),
                 p['fc2_W'], r2(p['fc2_b']), w3, r2(b3))
    return z[:, 0]


# speedup vs baseline: 2.4071x; 1.3657x over previous
import functools

import jax
import jax.numpy as jnp
from jax import lax
from jax.experimental import pallas as pl
from jax.experimental.pallas import tpu as pltpu
from jax.experimental.pallas import tpu_sc as plsc

N = 10000
E = 20000
B = 512
N3 = 15000
A = 45000
E3 = 60000

NW = 32
CHUNK = 128

N_PAD = 10112
N3_PAD = 15104
B_PAD = 640
E_PAD = 20480
A_PAD = 45056
E3_PAD = 61440
NV_PAD = 12288
N3V_PAD = 16384


def _mesh():
    return plsc.VectorSubcoreMesh(core_axis_name="c", subcore_axis_name="s")


def _sc_gather(table, idx2, d):
    ep = idx2.shape[0] * idx2.shape[1] * CHUNK
    nch = ep // (NW * CHUNK)
    ept = ep // NW

    def body(tab_hbm, idx_hbm, out_hbm, idxv, rowsv, sem):
        c = lax.axis_index("c")
        s = lax.axis_index("s")
        wid = s * 2 + c
        pltpu.sync_copy(idx_hbm.at[wid], idxv)
        descs = [pltpu.async_copy(tab_hbm.at[idxv.at[j]],
                                  rowsv.at[pl.ds(j * CHUNK, CHUNK)], sem)
                 for j in range(nch)]
        for dsc in descs:
            dsc.wait()
        pltpu.sync_copy(rowsv, out_hbm.at[pl.ds(wid * ept, ept)])

    f = pl.kernel(
        body,
        out_type=jax.ShapeDtypeStruct((ep, d), jnp.float32),
        mesh=_mesh(),
        compiler_params=pltpu.CompilerParams(use_tc_tiling_on_sc=False),
        scratch_types=[
            pltpu.VMEM((nch, CHUNK), jnp.int32),
            pltpu.VMEM((ept, d), jnp.float32),
            pltpu.SemaphoreType.DMA,
        ],
    )
    return f(table, idx2)


def _sc_scatter_add(vals, idx2, n_acc, d):
    ep = vals.shape[0]
    nch = ep // (NW * CHUNK)
    ept = ep // NW
    zr = n_acc // 16
    zeros = jnp.zeros((zr, d), jnp.float32)

    def body(vals_hbm, idx_hbm, z_hbm, out_hbm, idxv, valsv, acc, sem):
        c = lax.axis_index("c")
        s = lax.axis_index("s")
        wid = s * 2 + c
        pltpu.sync_copy(z_hbm, acc.at[pl.ds(s * zr, zr)])
        plsc.subcore_barrier()
        pltpu.sync_copy(idx_hbm.at[wid], idxv)
        pltpu.sync_copy(vals_hbm.at[pl.ds(wid * ept, ept)], valsv)
        descs = [pltpu.async_copy(valsv.at[pl.ds(j * CHUNK, CHUNK)],
                                  acc.at[idxv.at[j]], sem, add=True)
                 for j in range(nch)]
        for dsc in descs:
            dsc.wait()
        plsc.subcore_barrier()
        pltpu.sync_copy(acc.at[pl.ds(s * zr, zr)],
                        out_hbm.at[pl.ds(c * n_acc + s * zr, zr)])

    f = pl.kernel(
        body,
        out_type=jax.ShapeDtypeStruct((2 * n_acc, d), jnp.float32),
        mesh=_mesh(),
        compiler_params=pltpu.CompilerParams(use_tc_tiling_on_sc=False),
        scratch_types=[
            pltpu.VMEM((nch, CHUNK), jnp.int32),
            pltpu.VMEM((ept, d), jnp.float32),
            pltpu.VMEM_SHARED((n_acc, d), jnp.float32),
            pltpu.SemaphoreType.DMA,
        ],
    )
    return f(vals, idx2, zeros).reshape(2, n_acc, d)


def _sc_gather_scatter_add(table, src2, dst2, n_acc, d):
    ep = src2.shape[0] * src2.shape[1] * CHUNK
    nch = ep // (NW * CHUNK)
    zr = n_acc // 16
    zeros = jnp.zeros((zr, d), jnp.float32)

    nbuf = 4

    def body(tab_hbm, s_hbm, d_hbm, z_hbm, out_hbm, sidx, didx, rowsv, acc,
             sem0, sem1, sem2, sem3):
        sems = [sem0, sem1, sem2, sem3]
        c = lax.axis_index("c")
        s = lax.axis_index("s")
        wid = s * 2 + c
        pltpu.sync_copy(z_hbm, acc.at[pl.ds(s * zr, zr)])
        plsc.subcore_barrier()
        pltpu.sync_copy(s_hbm.at[wid], sidx)
        pltpu.sync_copy(d_hbm.at[wid], didx)
        gd = [None] * nch
        for j in range(min(nbuf, nch)):
            gd[j] = pltpu.async_copy(tab_hbm.at[sidx.at[j]], rowsv.at[j % nbuf],
                                     sems[j % nbuf])
        for j in range(nch):
            b = j % nbuf
            gd[j].wait()
            pltpu.sync_copy(rowsv.at[b], acc.at[didx.at[j]], add=True)
            nj = j + nbuf
            if nj < nch:
                gd[nj] = pltpu.async_copy(tab_hbm.at[sidx.at[nj]],
                                          rowsv.at[b], sems[b])
        plsc.subcore_barrier()
        pltpu.sync_copy(acc.at[pl.ds(s * zr, zr)],
                        out_hbm.at[pl.ds(c * n_acc + s * zr, zr)])

    f = pl.kernel(
        body,
        out_type=jax.ShapeDtypeStruct((2 * n_acc, d), jnp.float32),
        mesh=_mesh(),
        compiler_params=pltpu.CompilerParams(use_tc_tiling_on_sc=False),
        scratch_types=[
            pltpu.VMEM((nch, CHUNK), jnp.int32),
            pltpu.VMEM((nch, CHUNK), jnp.int32),
            pltpu.VMEM((nbuf, CHUNK, d), jnp.float32),
            pltpu.VMEM_SHARED((n_acc, d), jnp.float32),
            pltpu.SemaphoreType.DMA,
            pltpu.SemaphoreType.DMA,
            pltpu.SemaphoreType.DMA,
            pltpu.SemaphoreType.DMA,
        ],
    )
    return f(table, src2, dst2, zeros).reshape(2, n_acc, d)


def _tc_edge_messages(ea, xs, w1x, b1x, w2x, sel, mi, mo):
    ep = ea.shape[0]
    be = 256
    grid = ep // be
    w = mi * mo

    def body(ea_ref, xs_ref, w1_ref, b1_ref, w2_ref, sel_ref, out_ref):
        g = jnp.maximum(
            jnp.dot(ea_ref[:], w1_ref[:], preferred_element_type=jnp.float32)
            + b1_ref[:], 0.0)
        g2 = jnp.dot(g, w2_ref[:], preferred_element_type=jnp.float32)
        xt = jnp.tile(xs_ref[:], (1, mo))
        out_ref[:] = jnp.dot(xt * g2, sel_ref[:],
                             preferred_element_type=jnp.float32)

    return pl.pallas_call(
        body,
        grid=(grid,),
        in_specs=[
            pl.BlockSpec((be, 8), lambda i: (i, 0)),
            pl.BlockSpec((be, mi), lambda i: (i, 0)),
            pl.BlockSpec((8, 136), lambda i: (0, 0)),
            pl.BlockSpec((1, 136), lambda i: (0, 0)),
            pl.BlockSpec((136, w), lambda i: (0, 0)),
            pl.BlockSpec((w, mo), lambda i: (0, 0)),
        ],
        out_specs=pl.BlockSpec((be, mo), lambda i: (i, 0)),
        out_shape=jax.ShapeDtypeStruct((ep, mo), jnp.float32),
    )(ea, xs, w1x, b1x, w2x, sel)


def _tc_nnconv_combine(agg, hprev, root, bias):
    np_, mi = hprev.shape
    mo = root.shape[1]
    bn = 1024
    grid = pl.cdiv(np_, bn)

    def body(a_ref, h_ref, r_ref, b_ref, o_ref):
        o_ref[:] = jnp.maximum(
            a_ref[0] + a_ref[1]
            + jnp.dot(h_ref[:], r_ref[:], preferred_element_type=jnp.float32)
            + b_ref[:], 0.0)

    return pl.pallas_call(
        body,
        grid=(grid,),
        in_specs=[
            pl.BlockSpec((2, bn, mo), lambda i: (0, i, 0)),
            pl.BlockSpec((bn, mi), lambda i: (i, 0)),
            pl.BlockSpec((mi, mo), lambda i: (0, 0)),
            pl.BlockSpec((1, mo), lambda i: (0, 0)),
        ],
        out_specs=pl.BlockSpec((bn, mo), lambda i: (i, 0)),
        out_shape=jax.ShapeDtypeStruct((np_, mo), jnp.float32),
    )(agg, hprev, root, bias)


def _tc_pool_finalize(pooled, cnt, iso):
    np_ = iso.shape[0]
    bn = 1024
    grid = pl.cdiv(np_, bn)

    def body(p_ref, c_ref, iso_ref, o_ref):
        c = c_ref[0][:, 0:1] + c_ref[1][:, 0:1]
        pool = (p_ref[0] + p_ref[1]) / jnp.maximum(c, 1.0)
        o_ref[:] = jnp.concatenate([pool, iso_ref[:]], axis=1)

    return pl.pallas_call(
        body,
        grid=(grid,),
        in_specs=[
            pl.BlockSpec((2, bn, 64), lambda i: (0, i, 0)),
            pl.BlockSpec((2, bn, 16), lambda i: (0, i, 0)),
            pl.BlockSpec((bn, 64), lambda i: (i, 0)),
        ],
        out_specs=pl.BlockSpec((bn, 128), lambda i: (i, 0)),
        out_shape=jax.ShapeDtypeStruct((np_, 128), jnp.float32),
    )(pooled, cnt, iso)


def _tc_gc_combine(aggs, h, wrel, brel, wroot):
    np_, mi = h.shape
    mo = wrel.shape[1]
    bn = 1024
    grid = pl.cdiv(np_, bn)

    def body(*refs):
        a_refs = refs[:len(aggs)]
        h_ref, wr_ref, br_ref, wo_ref, o_ref = refs[len(aggs):]
        a = jnp.concatenate([r[0] + r[1] for r in a_refs], axis=1)
        o_ref[:] = jnp.maximum(
            jnp.dot(a, wr_ref[:], preferred_element_type=jnp.float32)
            + br_ref[:]
            + jnp.dot(h_ref[:], wo_ref[:], preferred_element_type=jnp.float32),
            0.0)

    return pl.pallas_call(
        body,
        grid=(grid,),
        in_specs=[
            pl.BlockSpec((2, bn, a.shape[2]), lambda i: (0, i, 0))
            for a in aggs
        ] + [
            pl.BlockSpec((bn, mi), lambda i: (i, 0)),
            pl.BlockSpec((mi, mo), lambda i: (0, 0)),
            pl.BlockSpec((1, mo), lambda i: (0, 0)),
            pl.BlockSpec((mi, mo), lambda i: (0, 0)),
        ],
        out_specs=pl.BlockSpec((bn, mo), lambda i: (i, 0)),
        out_shape=jax.ShapeDtypeStruct((np_, mo), jnp.float32),
    )(*aggs, h, wrel, brel, wroot)


def _tc_head(x1p, x3p, w1, b1, w2, b2, w3, b3):
    def body(x1_ref, x3_ref, w1_ref, b1_ref, w2_ref, b2_ref, w3_ref, b3_ref,
             o_ref):
        z = jnp.concatenate([x1_ref[0] + x1_ref[1], x3_ref[0] + x3_ref[1]],
                            axis=1)
        z = jnp.maximum(
            jnp.dot(z, w1_ref[:], preferred_element_type=jnp.float32)
            + b1_ref[:], 0.0)
        z = jnp.maximum(
            jnp.dot(z, w2_ref[:], preferred_element_type=jnp.float32)
            + b2_ref[:], 0.0)
        o_ref[:] = (jnp.dot(z, w3_ref[:], preferred_element_type=jnp.float32)
                    + b3_ref[:])

    return pl.pallas_call(
        body,
        grid=(1,),
        in_specs=[
            pl.BlockSpec((2, B, 64), lambda i: (0, 0, 0)),
            pl.BlockSpec((2, B, 64), lambda i: (0, 0, 0)),
            pl.BlockSpec((128, 64), lambda i: (0, 0)),
            pl.BlockSpec((1, 64), lambda i: (0, 0)),
            pl.BlockSpec((64, 32), lambda i: (0, 0)),
            pl.BlockSpec((1, 32), lambda i: (0, 0)),
            pl.BlockSpec((32, 8), lambda i: (0, 0)),
            pl.BlockSpec((1, 8), lambda i: (0, 0)),
        ],
        out_specs=pl.BlockSpec((B, 8), lambda i: (0, 0)),
        out_shape=jax.ShapeDtypeStruct((B, 8), jnp.float32),
    )(x1p, x3p, w1, b1, w2, b2, w3, b3)


def _pad_rows(a, n):
    return jnp.pad(a, ((0, n - a.shape[0]), (0, 0)))


def _pad_idx2(idx, ep, fill):
    return jnp.pad(idx, (0, ep - idx.shape[0]),
                   constant_values=fill).reshape(NW, ep // (NW * CHUNK), CHUNK)


def kernel(x, edge_index, edge_attr, batch, assignment_index_3, iso_type_3,
           edge_index_3, batch_3, params):
    p = params

    x_p = _pad_rows(x, N_PAD)
    iso_p = _pad_rows(iso_type_3, N3_PAD)
    ea_p = jnp.pad(edge_attr, ((0, E_PAD - E), (0, 1)))

    src2 = _pad_idx2(edge_index[0], E_PAD, 0)
    dst2 = _pad_idx2(edge_index[1], E_PAD, N)
    row2 = _pad_idx2(assignment_index_3[0], A_PAD, 0)
    col2 = _pad_idx2(assignment_index_3[1], A_PAD, N3)
    src3_2 = _pad_idx2(edge_index_3[0], E3_PAD, 0)
    dst3_2 = _pad_idx2(edge_index_3[1], E3_PAD, N3)
    batch2 = _pad_idx2(batch, NV_PAD, B)
    batch3_2 = _pad_idx2(batch_3, N3V_PAD, B)

    def r2(b):
        return b.reshape(1, -1)

    h = x_p
    specs = [
        (16, 32, 'nn1', 'conv1'),
        (32, 64, 'nn2', 'conv2'),
        (64, 64, 'nn3', 'conv3'),
    ]
    for mi, mo, nn, conv in specs:
        w1x = jnp.pad(p[nn + '_W1'], ((0, 1), (0, 8)))
        b1x = jnp.concatenate([p[nn + '_b1'], jnp.ones((8,), jnp.float32)]
                              ).reshape(1, 136)
        w2perm = p[nn + '_W2'].reshape(128, mi, mo).transpose(0, 2, 1)
        w2perm = w2perm.reshape(128, mi * mo)
        b2perm = p[nn + '_b2'].reshape(mi, mo).T.reshape(1, mi * mo)
        w2x = jnp.concatenate(
            [w2perm, b2perm, jnp.zeros((7, mi * mo), jnp.float32)], axis=0
        )
        sel = jnp.repeat(jnp.eye(mo, dtype=jnp.float32), mi, axis=0)
        xs = _sc_gather(h, src2, mi)
        msg = _tc_edge_messages(ea_p, xs, w1x, b1x, w2x, sel, mi, mo)
        agg = _sc_scatter_add(msg, dst2, N_PAD, mo)
        h = _tc_nnconv_combine(agg, h, p[conv + '_root'],
                               r2(p[conv + '_bias']))

    x1p = _sc_scatter_add(_pad_rows(h, NV_PAD), batch2, B_PAD, 64)[:, :B]

    pooled = _sc_gather_scatter_add(h, row2, col2, N3_PAD, 64)
    ones16 = jnp.ones((A_PAD, 16), jnp.float32)
    cnt = _sc_scatter_add(ones16, col2, N3_PAD, 16)
    h3 = _tc_pool_finalize(pooled, cnt, iso_p)

    agg_lo = _sc_gather_scatter_add(h3[:, :64], src3_2, dst3_2, N3_PAD, 64)
    agg_hi = _sc_gather_scatter_add(h3[:, 64:], src3_2, dst3_2, N3_PAD, 64)
    h3 = _tc_gc_combine([agg_lo, agg_hi], h3, p['conv6_Wrel'],
                        r2(p['conv6_brel']), p['conv6_Wroot'])
    agg3 = _sc_gather_scatter_add(h3, src3_2, dst3_2, N3_PAD, 64)
    h3 = _tc_gc_combine([agg3], h3, p['conv7_Wrel'], r2(p['conv7_brel']),
                        p['conv7_Wroot'])

    x3p = _sc_scatter_add(_pad_rows(h3, N3V_PAD), batch3_2, B_PAD, 64)[:, :B]

    w3 = jnp.pad(p['fc3_W'], ((0, 0), (0, 7)))
    b3 = jnp.pad(p['fc3_b'], (0, 7))
    z = _tc_head(x1p, x3p, p['fc1_W'], r2(p['fc1_b']),
                 p['fc2_W'], r2(p['fc2_b']), w3, r2(b3))
    return z[:, 0]


# speedup vs baseline: 2.4820x; 1.0311x over previous
import functools

import jax
import jax.numpy as jnp
from jax import lax
from jax.experimental import pallas as pl
from jax.experimental.pallas import tpu as pltpu
from jax.experimental.pallas import tpu_sc as plsc

N = 10000
E = 20000
B = 512
N3 = 15000
A = 45000
E3 = 60000

NW = 32
CHUNK = 128

N_PAD = 10112
N3_PAD = 15104
B_PAD = 640
E_PAD = 20480
A_PAD = 45056
E3_PAD = 61440
NV_PAD = 12288
N3V_PAD = 16384


def _mesh():
    return plsc.VectorSubcoreMesh(core_axis_name="c", subcore_axis_name="s")


def _sc_gather(table, idx2, d):
    ep = idx2.shape[0] * idx2.shape[1] * CHUNK
    nch = ep // (NW * CHUNK)
    ept = ep // NW

    def body(tab_hbm, idx_hbm, out_hbm, idxv, rowsv, sem):
        c = lax.axis_index("c")
        s = lax.axis_index("s")
        wid = s * 2 + c
        pltpu.sync_copy(idx_hbm.at[wid], idxv)
        descs = [pltpu.async_copy(tab_hbm.at[idxv.at[j]],
                                  rowsv.at[pl.ds(j * CHUNK, CHUNK)], sem)
                 for j in range(nch)]
        for dsc in descs:
            dsc.wait()
        pltpu.sync_copy(rowsv, out_hbm.at[pl.ds(wid * ept, ept)])

    f = pl.kernel(
        body,
        out_type=jax.ShapeDtypeStruct((ep, d), jnp.float32),
        mesh=_mesh(),
        compiler_params=pltpu.CompilerParams(use_tc_tiling_on_sc=False),
        scratch_types=[
            pltpu.VMEM((nch, CHUNK), jnp.int32),
            pltpu.VMEM((ept, d), jnp.float32),
            pltpu.SemaphoreType.DMA,
        ],
    )
    return f(table, idx2)


def _sc_scatter_add(vals, idx2, n_acc, d, col_off=0):
    ep = vals.shape[0]
    nch = ep // (NW * CHUNK)
    ept = ep // NW
    zr = n_acc // 16
    zeros = jnp.zeros((zr, d), jnp.float32)

    def body(vals_hbm, idx_hbm, z_hbm, out_hbm, idxv, valsv, acc, sem):
        c = lax.axis_index("c")
        s = lax.axis_index("s")
        wid = s * 2 + c
        pltpu.sync_copy(z_hbm, acc.at[pl.ds(s * zr, zr)])
        plsc.subcore_barrier()
        pltpu.sync_copy(idx_hbm.at[wid], idxv)
        pltpu.sync_copy(vals_hbm.at[pl.ds(wid * ept, ept),
                                    pl.ds(col_off, d)], valsv)
        descs = [pltpu.async_copy(valsv.at[pl.ds(j * CHUNK, CHUNK)],
                                  acc.at[idxv.at[j]], sem, add=True)
                 for j in range(nch)]
        for dsc in descs:
            dsc.wait()
        plsc.subcore_barrier()
        pltpu.sync_copy(acc.at[pl.ds(s * zr, zr)],
                        out_hbm.at[pl.ds(c * n_acc + s * zr, zr)])

    f = pl.kernel(
        body,
        out_type=jax.ShapeDtypeStruct((2 * n_acc, d), jnp.float32),
        mesh=_mesh(),
        compiler_params=pltpu.CompilerParams(use_tc_tiling_on_sc=False),
        scratch_types=[
            pltpu.VMEM((nch, CHUNK), jnp.int32),
            pltpu.VMEM((ept, d), jnp.float32),
            pltpu.VMEM_SHARED((n_acc, d), jnp.float32),
            pltpu.SemaphoreType.DMA,
        ],
    )
    return f(vals, idx2, zeros).reshape(2, n_acc, d)


def _sc_gather_scatter_add(table, src2, dst2, n_acc, d):
    ep = src2.shape[0] * src2.shape[1] * CHUNK
    nch = ep // (NW * CHUNK)
    zr = n_acc // 16
    zeros = jnp.zeros((zr, d), jnp.float32)

    nbuf = 4

    def body(tab_hbm, s_hbm, d_hbm, z_hbm, out_hbm, sidx, didx, rowsv, acc,
             sem0, sem1, sem2, sem3):
        sems = [sem0, sem1, sem2, sem3]
        c = lax.axis_index("c")
        s = lax.axis_index("s")
        wid = s * 2 + c
        pltpu.sync_copy(z_hbm, acc.at[pl.ds(s * zr, zr)])
        plsc.subcore_barrier()
        pltpu.sync_copy(s_hbm.at[wid], sidx)
        pltpu.sync_copy(d_hbm.at[wid], didx)
        gd = [None] * nch
        for j in range(min(nbuf, nch)):
            gd[j] = pltpu.async_copy(tab_hbm.at[sidx.at[j]], rowsv.at[j % nbuf],
                                     sems[j % nbuf])
        for j in range(nch):
            b = j % nbuf
            gd[j].wait()
            pltpu.sync_copy(rowsv.at[b], acc.at[didx.at[j]], add=True)
            nj = j + nbuf
            if nj < nch:
                gd[nj] = pltpu.async_copy(tab_hbm.at[sidx.at[nj]],
                                          rowsv.at[b], sems[b])
        plsc.subcore_barrier()
        pltpu.sync_copy(acc.at[pl.ds(s * zr, zr)],
                        out_hbm.at[pl.ds(c * n_acc + s * zr, zr)])

    f = pl.kernel(
        body,
        out_type=jax.ShapeDtypeStruct((2 * n_acc, d), jnp.float32),
        mesh=_mesh(),
        compiler_params=pltpu.CompilerParams(use_tc_tiling_on_sc=False),
        scratch_types=[
            pltpu.VMEM((nch, CHUNK), jnp.int32),
            pltpu.VMEM((nch, CHUNK), jnp.int32),
            pltpu.VMEM((nbuf, CHUNK, d), jnp.float32),
            pltpu.VMEM_SHARED((n_acc, d), jnp.float32),
            pltpu.SemaphoreType.DMA,
            pltpu.SemaphoreType.DMA,
            pltpu.SemaphoreType.DMA,
            pltpu.SemaphoreType.DMA,
        ],
    )
    return f(table, src2, dst2, zeros).reshape(2, n_acc, d)


def _sc_scatter_combine(msg, idx16, hroot, n_acc, d):
    ep = msg.shape[0]
    d2 = d // 2
    nch = ep // (16 * CHUNK)
    ept = ep // 16
    zr = n_acc // 16

    def body(msg_hbm, idx_hbm, hr_hbm, out_hbm, idxv, valsv, acc, sem):
        c = lax.axis_index("c")
        s = lax.axis_index("s")
        pltpu.sync_copy(hr_hbm.at[pl.ds(s * zr, zr), pl.ds(c * d2, d2)],
                        acc.at[pl.ds(s * zr, zr)])
        plsc.subcore_barrier()
        pltpu.sync_copy(idx_hbm.at[s], idxv)
        pltpu.sync_copy(msg_hbm.at[pl.ds(s * ept, ept), pl.ds(c * d2, d2)],
                        valsv)
        descs = [pltpu.async_copy(valsv.at[pl.ds(j * CHUNK, CHUNK)],
                                  acc.at[idxv.at[j]], sem, add=True)
                 for j in range(nch)]
        for dsc in descs:
            dsc.wait()
        plsc.subcore_barrier()
        pltpu.sync_copy(acc.at[pl.ds(s * zr, zr)],
                        out_hbm.at[pl.ds(s * zr, zr), pl.ds(c * d2, d2)])

    f = pl.kernel(
        body,
        out_type=jax.ShapeDtypeStruct((n_acc, d), jnp.float32),
        mesh=_mesh(),
        compiler_params=pltpu.CompilerParams(use_tc_tiling_on_sc=False),
        scratch_types=[
            pltpu.VMEM((nch, CHUNK), jnp.int32),
            pltpu.VMEM((ept, d2), jnp.float32),
            pltpu.VMEM_SHARED((n_acc, d2), jnp.float32),
            pltpu.SemaphoreType.DMA,
        ],
    )
    return f(msg, idx16, hroot)


def _tc_edge_messages(ea, xs, w1x, b1x, w2x, sel, mi, mo, relu_xs):
    ep = ea.shape[0]
    be = 256
    grid = ep // be
    w = mi * mo

    def body(ea_ref, xs_ref, w1_ref, b1_ref, w2_ref, sel_ref, out_ref):
        g = jnp.maximum(
            jnp.dot(ea_ref[:], w1_ref[:], preferred_element_type=jnp.float32)
            + b1_ref[:], 0.0)
        g2 = jnp.dot(g, w2_ref[:], preferred_element_type=jnp.float32)
        xsv = jnp.maximum(xs_ref[:], 0.0) if relu_xs else xs_ref[:]
        xt = jnp.tile(xsv, (1, mo))
        out_ref[:] = jnp.dot(xt * g2, sel_ref[:],
                             preferred_element_type=jnp.float32)

    return pl.pallas_call(
        body,
        grid=(grid,),
        in_specs=[
            pl.BlockSpec((be, 8), lambda i: (i, 0)),
            pl.BlockSpec((be, mi), lambda i: (i, 0)),
            pl.BlockSpec((8, 136), lambda i: (0, 0)),
            pl.BlockSpec((1, 136), lambda i: (0, 0)),
            pl.BlockSpec((136, w), lambda i: (0, 0)),
            pl.BlockSpec((w, mo), lambda i: (0, 0)),
        ],
        out_specs=pl.BlockSpec((be, mo), lambda i: (i, 0)),
        out_shape=jax.ShapeDtypeStruct((ep, mo), jnp.float32),
    )(ea, xs, w1x, b1x, w2x, sel)


def _tc_hroot(hprev, root, bias, relu_in):
    np_, mi = hprev.shape
    mo = root.shape[1]
    bn = 1024
    grid = pl.cdiv(np_, bn)

    def body(h_ref, r_ref, b_ref, o_ref):
        h = jnp.maximum(h_ref[:], 0.0) if relu_in else h_ref[:]
        o_ref[:] = (jnp.dot(h, r_ref[:], preferred_element_type=jnp.float32)
                    + b_ref[:])

    return pl.pallas_call(
        body,
        grid=(grid,),
        in_specs=[
            pl.BlockSpec((bn, mi), lambda i: (i, 0)),
            pl.BlockSpec((mi, mo), lambda i: (0, 0)),
            pl.BlockSpec((1, mo), lambda i: (0, 0)),
        ],
        out_specs=pl.BlockSpec((bn, mo), lambda i: (i, 0)),
        out_shape=jax.ShapeDtypeStruct((np_, mo), jnp.float32),
    )(hprev, root, bias)


def _tc_relu_ones(h_pre, n_out):
    np_, d = h_pre.shape
    bn = 1024
    grid = pl.cdiv(np_, bn)

    def body(h_ref, o_ref):
        o_ref[:] = jnp.concatenate(
            [jnp.maximum(h_ref[:], 0.0), jnp.ones((bn, 16), jnp.float32)],
            axis=1)

    return pl.pallas_call(
        body,
        grid=(grid,),
        in_specs=[pl.BlockSpec((bn, d), lambda i: (i, 0))],
        out_specs=pl.BlockSpec((bn, d + 16), lambda i: (i, 0)),
        out_shape=jax.ShapeDtypeStruct((n_out, d + 16), jnp.float32),
    )(h_pre)


def _tc_pool_div(pooled):
    np_ = pooled.shape[1]
    bn = 1024
    grid = pl.cdiv(np_, bn)

    def body(p_ref, o_ref):
        s = p_ref[0] + p_ref[1]
        o_ref[:] = s[:, :64] / jnp.maximum(s[:, 64:65], 1.0)

    return pl.pallas_call(
        body,
        grid=(grid,),
        in_specs=[pl.BlockSpec((2, bn, 80), lambda i: (0, i, 0))],
        out_specs=pl.BlockSpec((bn, 64), lambda i: (i, 0)),
        out_shape=jax.ShapeDtypeStruct((np_, 64), jnp.float32),
    )(pooled)


def _tc_gc_combine(aggs, hs, wrel, brel, wroot):
    np_ = hs[0].shape[0]
    mi = wrel.shape[0]
    mo = wrel.shape[1]
    bn = 1024
    grid = pl.cdiv(np_, bn)

    def body(*refs):
        a_refs = refs[:len(aggs)]
        h_refs = refs[len(aggs):len(aggs) + len(hs)]
        wr_ref, br_ref, wo_ref, o_ref = refs[len(aggs) + len(hs):]
        a = jnp.concatenate([r[0] + r[1] for r in a_refs], axis=1)
        h = jnp.concatenate([r[:] for r in h_refs], axis=1)
        o_ref[:] = jnp.maximum(
            jnp.dot(a, wr_ref[:], preferred_element_type=jnp.float32)
            + br_ref[:]
            + jnp.dot(h, wo_ref[:], preferred_element_type=jnp.float32),
            0.0)

    return pl.pallas_call(
        body,
        grid=(grid,),
        in_specs=[
            pl.BlockSpec((2, bn, a.shape[2]), lambda i: (0, i, 0))
            for a in aggs
        ] + [
            pl.BlockSpec((bn, hh.shape[1]), lambda i: (i, 0)) for hh in hs
        ] + [
            pl.BlockSpec((mi, mo), lambda i: (0, 0)),
            pl.BlockSpec((1, mo), lambda i: (0, 0)),
            pl.BlockSpec((mi, mo), lambda i: (0, 0)),
        ],
        out_specs=pl.BlockSpec((bn, mo), lambda i: (i, 0)),
        out_shape=jax.ShapeDtypeStruct((np_, mo), jnp.float32),
    )(*aggs, *hs, wrel, brel, wroot)


def _tc_head(x1p, x3p, w1, b1, w2, b2, w3, b3):
    def body(x1_ref, x3_ref, w1_ref, b1_ref, w2_ref, b2_ref, w3_ref, b3_ref,
             o_ref):
        z = jnp.concatenate([x1_ref[0] + x1_ref[1], x3_ref[0] + x3_ref[1]],
                            axis=1)
        z = jnp.maximum(
            jnp.dot(z, w1_ref[:], preferred_element_type=jnp.float32)
            + b1_ref[:], 0.0)
        z = jnp.maximum(
            jnp.dot(z, w2_ref[:], preferred_element_type=jnp.float32)
            + b2_ref[:], 0.0)
        o_ref[:] = (jnp.dot(z, w3_ref[:], preferred_element_type=jnp.float32)
                    + b3_ref[:])

    return pl.pallas_call(
        body,
        grid=(1,),
        in_specs=[
            pl.BlockSpec((2, B, 64), lambda i: (0, 0, 0)),
            pl.BlockSpec((2, B, 64), lambda i: (0, 0, 0)),
            pl.BlockSpec((128, 64), lambda i: (0, 0)),
            pl.BlockSpec((1, 64), lambda i: (0, 0)),
            pl.BlockSpec((64, 32), lambda i: (0, 0)),
            pl.BlockSpec((1, 32), lambda i: (0, 0)),
            pl.BlockSpec((32, 8), lambda i: (0, 0)),
            pl.BlockSpec((1, 8), lambda i: (0, 0)),
        ],
        out_specs=pl.BlockSpec((B, 8), lambda i: (0, 0)),
        out_shape=jax.ShapeDtypeStruct((B, 8), jnp.float32),
    )(x1p, x3p, w1, b1, w2, b2, w3, b3)


def _pad_rows(a, n):
    return jnp.pad(a, ((0, n - a.shape[0]), (0, 0)))


def _pad_idx2(idx, ep, fill):
    return jnp.pad(idx, (0, ep - idx.shape[0]),
                   constant_values=fill).reshape(NW, ep // (NW * CHUNK), CHUNK)


def _pad_idx16(idx, ep, fill):
    return jnp.pad(idx, (0, ep - idx.shape[0]),
                   constant_values=fill).reshape(16, ep // (16 * CHUNK), CHUNK)


def kernel(x, edge_index, edge_attr, batch, assignment_index_3, iso_type_3,
           edge_index_3, batch_3, params):
    p = params

    x_p = _pad_rows(x, N_PAD)
    iso_p = _pad_rows(iso_type_3, N3_PAD)
    ea_p = jnp.pad(edge_attr, ((0, E_PAD - E), (0, 1)))

    src2 = _pad_idx2(edge_index[0], E_PAD, 0)
    dst16 = _pad_idx16(edge_index[1], E_PAD, N)
    row2 = _pad_idx2(assignment_index_3[0], A_PAD, 0)
    col2 = _pad_idx2(assignment_index_3[1], A_PAD, N3)
    src3_2 = _pad_idx2(edge_index_3[0], E3_PAD, 0)
    dst3_2 = _pad_idx2(edge_index_3[1], E3_PAD, N3)
    batch2 = _pad_idx2(batch, NV_PAD, B)
    batch3_2 = _pad_idx2(batch_3, N3V_PAD, B)

    def r2(b):
        return b.reshape(1, -1)

    h = x_p
    specs = [
        (16, 32, 'nn1', 'conv1'),
        (32, 64, 'nn2', 'conv2'),
        (64, 64, 'nn3', 'conv3'),
    ]
    for li, (mi, mo, nn, conv) in enumerate(specs):
        relu_in = li > 0
        w1x = jnp.pad(p[nn + '_W1'], ((0, 1), (0, 8)))
        b1x = jnp.concatenate([p[nn + '_b1'], jnp.ones((8,), jnp.float32)]
                              ).reshape(1, 136)
        w2perm = p[nn + '_W2'].reshape(128, mi, mo).transpose(0, 2, 1)
        w2perm = w2perm.reshape(128, mi * mo)
        b2perm = p[nn + '_b2'].reshape(mi, mo).T.reshape(1, mi * mo)
        w2x = jnp.concatenate(
            [w2perm, b2perm, jnp.zeros((7, mi * mo), jnp.float32)], axis=0
        )
        sel = jnp.repeat(jnp.eye(mo, dtype=jnp.float32), mi, axis=0)
        xs = _sc_gather(h, src2, mi)
        hroot = _tc_hroot(h, p[conv + '_root'], r2(p[conv + '_bias']),
                          relu_in)
        msg = _tc_edge_messages(ea_p, xs, w1x, b1x, w2x, sel, mi, mo, relu_in)
        h = _sc_scatter_combine(msg, dst16, hroot, N_PAD, mo)

    h3f = _tc_relu_ones(h, NV_PAD)

    x1p = _sc_scatter_add(h3f, batch2, B_PAD, 64)[:, :B]

    pooled = _sc_gather_scatter_add(h3f, row2, col2, N3_PAD, 80)
    h3lo = _tc_pool_div(pooled)

    agg_lo = _sc_gather_scatter_add(h3lo, src3_2, dst3_2, N3_PAD, 64)
    agg_hi = _sc_gather_scatter_add(iso_p, src3_2, dst3_2, N3_PAD, 64)
    h3 = _tc_gc_combine([agg_lo, agg_hi], [h3lo, iso_p], p['conv6_Wrel'],
                        r2(p['conv6_brel']), p['conv6_Wroot'])
    agg3 = _sc_gather_scatter_add(h3, src3_2, dst3_2, N3_PAD, 64)
    h3 = _tc_gc_combine([agg3], [h3], p['conv7_Wrel'], r2(p['conv7_brel']),
                        p['conv7_Wroot'])

    x3p = _sc_scatter_add(_pad_rows(h3, N3V_PAD), batch3_2, B_PAD, 64)[:, :B]

    w3 = jnp.pad(p['fc3_W'], ((0, 0), (0, 7)))
    b3 = jnp.pad(p['fc3_b'], (0, 7))
    z = _tc_head(x1p, x3p, p['fc1_W'], r2(p['fc1_b']),
                 p['fc2_W'], r2(p['fc2_b']), w3, r2(b3))
    return z[:, 0]
